# baseline (speedup 1.0000x reference)
import functools

import jax
import jax.numpy as jnp
from jax import lax
from jax.experimental import pallas as pl
from jax.experimental.pallas import tpu as pltpu
from jax.experimental.pallas import tpu_sc as plsc

D = 64
LANES = 16
CW = 512
NCHUNK = 61
RANGE = NCHUNK * CW
MCAP = 1040
WCAP = 144
V_ENT = 1000000
TAIL0 = 32 * RANGE
TAIL1 = TAIL0 + CW

_CP = pltpu.CompilerParams(needs_layout_passes=False)


def _rsqrt(s):
    bi = lax.bitcast_convert_type(s, jnp.int32)
    bi = jnp.int32(0x5F3759DF) - (bi >> 1)
    y = lax.bitcast_convert_type(bi, jnp.float32)
    half = jnp.float32(0.5) * s
    for _ in range(3):
        y = y * (jnp.float32(1.5) - half * y * y)
    return y


def _make_extract_call(B):
    info = plsc.get_sparse_core_info()
    NC = info.num_cores
    mesh = plsc.VectorSubcoreMesh(core_axis_name="c", subcore_axis_name="s")
    n_scan = B // LANES
    out_len = B * D + D

    IB = 2048

    @functools.partial(
        pl.kernel,
        out_type=(jax.ShapeDtypeStruct((out_len,), jnp.float32),
                  jax.ShapeDtypeStruct((out_len,), jnp.float32)),
        mesh=mesh,
        compiler_params=_CP,
        scratch_types=[
            pltpu.VMEM((IB,), jnp.int32),
            pltpu.VMEM((MCAP,), jnp.int32),
            pltpu.VMEM((MCAP,), jnp.int32),
            pltpu.VMEM((MCAP,), jnp.int32),
            pltpu.VMEM((MCAP,), jnp.int32),
            pltpu.VMEM((WCAP,), jnp.int32),
            pltpu.VMEM((WCAP,), jnp.int32),
            pltpu.VMEM((D, CW), jnp.float32),
            pltpu.VMEM((D, CW), jnp.float32),
            pltpu.VMEM((D, D), jnp.float32),
            pltpu.VMEM((2, LANES * D), jnp.float32),
            pltpu.SemaphoreType.DMA,
            pltpu.SemaphoreType.DMA,
            pltpu.SemaphoreType.DMA,
            pltpu.SemaphoreType.DMA,
            pltpu.SemaphoreType.DMA,
        ],
    )
    def extract_call(h_hbm, t_hbm, entT_hbm, hx_hbm, tx_hbm,
                     idxbuf, mh_ent, mh_slot, mt_ent, mt_slot,
                     wl_col, wl_slot, cb0, cb1, tailbuf, rowbufs,
                     sem0, sem1, semr0, semr1, semi):
        wid = lax.axis_index("s") * NC + lax.axis_index("c")
        lo = wid * RANGE
        hi = lo + RANGE
        lanes = lax.iota(jnp.int32, LANES)
        trash = jnp.int32(B)

        sent = jnp.full((LANES,), jnp.int32(0x7FFFFFFF))
        strash = jnp.full((LANES,), trash)

        def prefill(i, c):
            sl = pl.ds(i * LANES, LANES)
            mh_ent[sl] = sent
            mt_ent[sl] = sent
            mh_slot[sl] = strash
            mt_slot[sl] = strash
            return c
        lax.fori_loop(0, MCAP // LANES, prefill, 0)

        is0 = wid == 0
        is1 = wid == 1

        def member(ev):
            m = (ev >= lo) & (ev < hi)
            m = m | (is0 & (ev >= TAIL0) & (ev < TAIL1))
            m = m | (is1 & (ev >= TAIL1))
            return m

        def prescan(idx_hbm, ment, mslot):
            def blk_body(b, pos):
                pltpu.async_copy(
                    idx_hbm.at[pl.ds(b * IB, IB)], idxbuf, semi).wait()

                def body(i, p):
                    ev = idxbuf[pl.ds(i * LANES, LANES)]
                    m = member(ev)
                    slots = b * IB + i * LANES + lanes
                    plsc.store_compressed(ment.at[pl.ds(p, LANES)], ev,
                                          mask=m)
                    plsc.store_compressed(mslot.at[pl.ds(p, LANES)], slots,
                                          mask=m)
                    cnt = plsc.all_reduce_population_count(m)
                    return p + cnt[0]
                return lax.fori_loop(0, IB // LANES, body, pos)
            return lax.fori_loop(0, B // IB, blk_body, jnp.int32(0))

        nh = prescan(h_hbm, mh_ent, mh_slot)
        nt = prescan(t_hbm, mt_ent, mt_slot)
        nh_g = (nh + LANES - 1) // LANES
        nt_g = (nt + LANES - 1) // LANES

        def drain_rb(rb_p):
            sem = semr0 if rb_p == 0 else semr1
            pltpu.make_async_copy(
                hx_hbm.at[pl.ds(0, LANES * D)], rowbufs.at[rb_p],
                sem).wait()

        def process_list(buf, cbase, cwidth, ment, mslot, n_g, out_ref,
                         rb_p, pend):
            def wpre(i, c):
                sl = pl.ds(i * LANES, LANES)
                wl_col[sl] = jnp.zeros((LANES,), jnp.int32)
                wl_slot[sl] = strash
                return c
            lax.fori_loop(0, WCAP // LANES, wpre, 0)

            def scan_body(i, pos):
                ev = ment[pl.ds(i * LANES, LANES)]
                sv = mslot[pl.ds(i * LANES, LANES)]
                m = (ev >= cbase) & (ev < cbase + cwidth)
                cols = ev - cbase
                plsc.store_compressed(wl_col.at[pl.ds(pos, LANES)], cols,
                                      mask=m)
                plsc.store_compressed(wl_slot.at[pl.ds(pos, LANES)], sv,
                                      mask=m)
                cnt = plsc.all_reduce_population_count(m)
                return pos + cnt[0]
            nw = lax.fori_loop(0, n_g, scan_body, jnp.int32(0))

            def grp_body(g, carry):
                @pl.when(carry > 0)
                def _():
                    drain_rb(rb_p)
                cols16 = wl_col[pl.ds(g * LANES, LANES)] & (cwidth - 1)
                slots16 = wl_slot[pl.ds(g * LANES, LANES)]
                for m in range(LANES):
                    cm = jnp.full((LANES,), cols16[m])
                    for k in range(D // LANES):
                        v = plsc.load_gather(buf, [k * LANES + lanes, cm])
                        rowbufs[rb_p, pl.ds(m * D + k * LANES, LANES)] = v
                semw = semr0 if rb_p == 0 else semr1
                for m in range(LANES):
                    pltpu.async_copy(
                        rowbufs.at[rb_p, pl.ds(m * D, D)],
                        out_ref.at[pl.ds(slots16[m] * D, D)], semw)
                return jnp.int32(1)
            return lax.fori_loop(0, (nw + LANES - 1) // LANES, grp_body,
                                 pend)

        def process_chunk(buf, cbase, cwidth, rb_p, pend):
            pend = process_list(buf, cbase, cwidth, mh_ent, mh_slot, nh_g,
                                hx_hbm, rb_p, pend)
            pend = process_list(buf, cbase, cwidth, mt_ent, mt_slot, nt_g,
                                tx_hbm, rb_p, pend)
            return pend

        def chunk_src(q):
            return entT_hbm.at[:, pl.ds(lo + q * CW, CW)]

        pltpu.async_copy(chunk_src(0), cb0, sem0)

        def pipe_body(qq, pends):
            p0, p1 = pends
            q0 = qq * 2
            pltpu.async_copy(chunk_src(q0 + 1), cb1, sem1)
            pltpu.make_async_copy(chunk_src(q0), cb0, sem0).wait()
            p0 = process_chunk(cb0, lo + q0 * CW, CW, 0, p0)
            pltpu.async_copy(chunk_src(q0 + 2), cb0, sem0)
            pltpu.make_async_copy(chunk_src(q0 + 1), cb1, sem1).wait()
            p1 = process_chunk(cb1, lo + (q0 + 1) * CW, CW, 1, p1)
            return (p0, p1)
        p0, p1 = lax.fori_loop(0, (NCHUNK - 1) // 2, pipe_body,
                               (jnp.int32(0), jnp.int32(0)))
        pltpu.make_async_copy(chunk_src(NCHUNK - 1), cb0, sem0).wait()
        p0 = process_chunk(cb0, lo + (NCHUNK - 1) * CW, CW, 0, p0)

        pltpu.async_copy(entT_hbm.at[:, pl.ds(TAIL0, CW)], cb1, sem1).wait()
        p1 = process_chunk(cb1, jnp.int32(TAIL0), CW, 1, p1)
        pltpu.async_copy(entT_hbm.at[:, pl.ds(TAIL1, D)], tailbuf,
                         sem1).wait()
        p0 = process_chunk(tailbuf, jnp.int32(TAIL1), D, 0, p0)

        @pl.when(p0 > 0)
        def _():
            drain_rb(0)

        @pl.when(p1 > 0)
        def _():
            drain_rb(1)

    return extract_call


def _make_score_call(B, VR):
    info = plsc.get_sparse_core_info()
    NC, NS = info.num_cores, info.num_subcores
    NW = NC * NS
    b_per_w = B // NW
    WAVE = b_per_w // 2
    mesh = plsc.VectorSubcoreMesh(core_axis_name="c", subcore_axis_name="s")
    wave_len = WAVE * D

    @functools.partial(
        pl.kernel,
        out_type=jax.ShapeDtypeStruct((B,), jnp.float32),
        mesh=mesh,
        compiler_params=_CP,
        scratch_types=[
            pltpu.VMEM((b_per_w,), jnp.int32),
            pltpu.VMEM((D, VR), jnp.float32),
            pltpu.VMEM((wave_len,), jnp.float32),
            pltpu.VMEM((wave_len,), jnp.float32),
            pltpu.VMEM((b_per_w,), jnp.float32),
            pltpu.SemaphoreType.DMA,
        ],
    )
    def score_call(r_hbm, relT_hbm, hx_hbm, tx_hbm, out_hbm,
                   ridx, relv, hflat, tflat, out_scr, sem):
        wid = lax.axis_index("s") * NC + lax.axis_index("c")
        base = wid * b_per_w
        lanes = lax.iota(jnp.int32, LANES)

        c1 = pltpu.async_copy(r_hbm.at[pl.ds(base, b_per_w)], ridx, sem)
        c2 = pltpu.async_copy(relT_hbm, relv, sem)
        c1.wait()
        c2.wait()

        zeros = jnp.zeros((LANES,), jnp.float32)

        for wave in range(2):
            wbase = base + wave * WAVE
            c3 = pltpu.async_copy(hx_hbm.at[pl.ds(wbase * D, wave_len)],
                                  hflat, sem)
            c4 = pltpu.async_copy(tx_hbm.at[pl.ds(wbase * D, wave_len)],
                                  tflat, sem)
            c3.wait()
            c4.wait()

            def group_body(g, c):
                fb = (g * LANES + lanes) * D
                re16 = ridx[pl.ds(wave * WAVE + g * LANES, LANES)]

                def sq_body(f4, accs):
                    ah, ar, at_ = accs
                    for u in range(4):
                        f = f4 * 4 + u
                        hv = plsc.load_gather(hflat, [fb + f])
                        tv = plsc.load_gather(tflat, [fb + f])
                        rv = plsc.load_gather(
                            relv, [jnp.full((LANES,), f), re16])
                        ah = ah + hv * hv
                        ar = ar + rv * rv
                        at_ = at_ + tv * tv
                    return ah, ar, at_

                sh, sr, st = lax.fori_loop(0, D // 4, sq_body,
                                           (zeros, zeros, zeros))
                ih, ir, it = _rsqrt(sh), _rsqrt(sr), _rsqrt(st)

                def sc_body(f4, acc):
                    for u in range(4):
                        f = f4 * 4 + u
                        hv = plsc.load_gather(hflat, [fb + f])
                        tv = plsc.load_gather(tflat, [fb + f])
                        rv = plsc.load_gather(
                            relv, [jnp.full((LANES,), f), re16])
                        acc = acc + jnp.abs(hv * ih + rv * ir - tv * it)
                    return acc

                sc = lax.fori_loop(0, D // 4, sc_body, zeros)
                out_scr[pl.ds(wave * WAVE + g * LANES, LANES)] = sc
                return c

            lax.fori_loop(0, WAVE // LANES, group_body, 0)

        pltpu.sync_copy(out_scr, out_hbm.at[pl.ds(base, b_per_w)])

    return score_call


def kernel(h, r, t, ent_table, rel_table):
    B = h.shape[0]
    VR = rel_table.shape[0]
    entT = ent_table.T
    relT = rel_table.T
    extract_call = _make_extract_call(B)
    hx, tx = extract_call(h.astype(jnp.int32), t.astype(jnp.int32), entT)
    score_call = _make_score_call(B, VR)
    score = score_call(r.astype(jnp.int32), relT, hx, tx)
    return score[:, None]
